# Initial kernel scaffold; baseline (speedup 1.0000x reference)
import jax
import jax.numpy as jnp
from jax.experimental import pallas as pl


def kernel(obs, neis, self_labels, nei_labels, modes, W_obs, b_obs, W_nei, b_nei, W_mode, b_mode):
    raise NotImplementedError("write your pallas kernel here")


# speedup vs baseline: 1.6534x; 1.6534x over previous
import jax
import jax.numpy as jnp
from jax import lax
from jax.experimental import pallas as pl

NUM_CLASS = 8
IN_SIZE = 2
OBS_LEN = 50
EMBED = 256
B = 1024
N = 32
NUM_MODES = 20
D_IN = IN_SIZE * OBS_LEN
D_PAD = 128

_BLK_A = 256
_BLK_B = 512


def _self_mode_body(obs_ref, lbl_ref, WoT_ref, b_obs_ref, W1_ref, W2_ref,
                    modes_ref, b_mode_ref, out_ref):
    lbl = lbl_ref[0, :, :]
    y_all = jnp.dot(obs_ref[...], WoT_ref[...],
                    preferred_element_type=jnp.float32)
    x = jnp.zeros((_BLK_A, EMBED), jnp.float32)
    for c in range(NUM_CLASS):
        feat = y_all[:, c * EMBED:(c + 1) * EMBED] + b_obs_ref[c, :][None, :]
        x = jnp.where(lbl == c, feat, x)
    xw1 = lax.dot_general(x, W1_ref[...], (((1,), (1,)), ((), ())),
                          preferred_element_type=jnp.float32)
    m2 = lax.dot_general(modes_ref[...], W2_ref[...], (((1,), (1,)), ((), ())),
                         preferred_element_type=jnp.float32)
    m2r = m2.reshape(NUM_CLASS, NUM_MODES * EMBED)
    onehot = (lbl ==
              lax.broadcasted_iota(jnp.int32, (_BLK_A, NUM_CLASS), 1)
              ).astype(jnp.float32)
    mode_part = jnp.dot(onehot, m2r,
                        preferred_element_type=jnp.float32)
    out = (mode_part.reshape(_BLK_A, NUM_MODES, EMBED)
           + xw1[:, None, :] + b_mode_ref[0, :][None, None, :])
    out_ref[...] = out


def _nei_body(neis_ref, lbl_ref, WnT_ref, b_nei_ref, out_ref):
    v = neis_ref[...]
    t = jnp.where(v >= 0, 1.0 / (v + 0.0001), 1.0 / (v - 0.0001))
    p = jnp.dot(t, WnT_ref[...],
                preferred_element_type=jnp.float32)
    lbl = lbl_ref[0, :, :]
    out = jnp.zeros((_BLK_B, EMBED), jnp.float32)
    for c in range(NUM_CLASS + 1):
        feat = p[:, c * EMBED:(c + 1) * EMBED] + b_nei_ref[c, :][None, :]
        out = jnp.where(lbl == c, feat, out)
    out_ref[...] = out


def kernel(obs, neis, self_labels, nei_labels, modes,
           W_obs, b_obs, W_nei, b_nei, W_mode, b_mode):
    obs_f = obs.reshape(B, D_IN)
    neis_f = neis.reshape(B * N, D_IN)
    obs_p = jnp.pad(obs_f, ((0, 0), (0, D_PAD - D_IN)))
    neis_p = jnp.pad(neis_f, ((0, 0), (0, D_PAD - D_IN)))
    WoT = jnp.pad(jnp.transpose(W_obs, (2, 0, 1)).reshape(D_IN, NUM_CLASS * EMBED),
                  ((0, D_PAD - D_IN), (0, 0)))
    WnT = jnp.pad(jnp.transpose(W_nei, (2, 0, 1)).reshape(D_IN, (NUM_CLASS + 1) * EMBED),
                  ((0, D_PAD - D_IN), (0, 0)))
    W1 = W_mode[:, :EMBED]
    W2 = W_mode[:, EMBED:]
    modes_flat = modes.reshape(NUM_CLASS * NUM_MODES, EMBED)
    b_mode2 = b_mode.reshape(1, EMBED)
    lbl_a = self_labels.reshape(B // _BLK_A, _BLK_A, 1)
    lbl_b = nei_labels.reshape(B * N // _BLK_B, _BLK_B, 1)

    n_a = B // _BLK_A
    x_out = pl.pallas_call(
        _self_mode_body,
        grid=(n_a,),
        in_specs=[
            pl.BlockSpec((_BLK_A, D_PAD), lambda i: (i, 0)),
            pl.BlockSpec((1, _BLK_A, 1), lambda i: (i, 0, 0)),
            pl.BlockSpec((D_PAD, NUM_CLASS * EMBED), lambda i: (0, 0)),
            pl.BlockSpec((NUM_CLASS, EMBED), lambda i: (0, 0)),
            pl.BlockSpec((EMBED, EMBED), lambda i: (0, 0)),
            pl.BlockSpec((EMBED, EMBED), lambda i: (0, 0)),
            pl.BlockSpec((NUM_CLASS * NUM_MODES, EMBED), lambda i: (0, 0)),
            pl.BlockSpec((1, EMBED), lambda i: (0, 0)),
        ],
        out_specs=pl.BlockSpec((_BLK_A, NUM_MODES, EMBED), lambda i: (i, 0, 0)),
        out_shape=jax.ShapeDtypeStruct((B, NUM_MODES, EMBED), jnp.float32),
    )(obs_p, lbl_a, WoT, b_obs, W1, W2, modes_flat, b_mode2)

    n_b = B * N // _BLK_B
    nei_out = pl.pallas_call(
        _nei_body,
        grid=(n_b,),
        in_specs=[
            pl.BlockSpec((_BLK_B, D_PAD), lambda i: (i, 0)),
            pl.BlockSpec((1, _BLK_B, 1), lambda i: (i, 0, 0)),
            pl.BlockSpec((D_PAD, (NUM_CLASS + 1) * EMBED), lambda i: (0, 0)),
            pl.BlockSpec((NUM_CLASS + 1, EMBED), lambda i: (0, 0)),
        ],
        out_specs=pl.BlockSpec((_BLK_B, EMBED), lambda i: (i, 0)),
        out_shape=jax.ShapeDtypeStruct((B * N, EMBED), jnp.float32),
    )(neis_p, lbl_b, WnT, b_nei)

    return (x_out, nei_out.reshape(B, N, EMBED))


# speedup vs baseline: 1.7868x; 1.0807x over previous
import jax
import jax.numpy as jnp
from jax import lax
from jax.experimental import pallas as pl

NUM_CLASS = 8
IN_SIZE = 2
OBS_LEN = 50
EMBED = 256
B = 1024
N = 32
NUM_MODES = 20
D_IN = IN_SIZE * OBS_LEN
D_PAD = 128

_BLK_A = 256
_BLK_B = 512


def _self_mode_body(obs_ref, lbl_ref, WoT_ref, b_obs_ref, W1_ref, W2_ref,
                    modes_ref, b_mode_ref, out_ref):
    lbl = lbl_ref[0, :, :]
    y_all = jnp.dot(obs_ref[...], WoT_ref[...],
                    preferred_element_type=jnp.float32)
    x = jnp.zeros((_BLK_A, EMBED), jnp.float32)
    for c in range(NUM_CLASS):
        feat = y_all[:, c * EMBED:(c + 1) * EMBED] + b_obs_ref[c, :][None, :]
        x = jnp.where(lbl == c, feat, x)
    xw1 = lax.dot_general(x, W1_ref[...], (((1,), (1,)), ((), ())),
                          preferred_element_type=jnp.float32)
    m2 = lax.dot_general(modes_ref[...], W2_ref[...], (((1,), (1,)), ((), ())),
                         preferred_element_type=jnp.float32)
    m2r = m2.reshape(NUM_CLASS, NUM_MODES * EMBED)
    onehot = (lbl ==
              lax.broadcasted_iota(jnp.int32, (_BLK_A, NUM_CLASS), 1)
              ).astype(jnp.float32)
    mode_part = jnp.dot(onehot, m2r,
                        preferred_element_type=jnp.float32)
    out = (mode_part.reshape(_BLK_A, NUM_MODES, EMBED)
           + xw1[:, None, :] + b_mode_ref[0, :][None, None, :])
    out_ref[...] = out


def _nei_body(neis_ref, lbl_ref, WnT_ref, b_nei_ref, out_ref):
    v = neis_ref[...]
    t = jnp.where(v >= 0, 1.0 / (v + 0.0001), 1.0 / (v - 0.0001))
    p = jnp.dot(t.astype(jnp.bfloat16), WnT_ref[...],
                preferred_element_type=jnp.float32)
    lbl = lbl_ref[0, :, :]
    out = jnp.zeros((_BLK_B, EMBED), jnp.float32)
    for c in range(NUM_CLASS + 1):
        feat = p[:, c * EMBED:(c + 1) * EMBED] + b_nei_ref[c, :][None, :]
        out = jnp.where(lbl == c, feat, out)
    out_ref[...] = out


def kernel(obs, neis, self_labels, nei_labels, modes,
           W_obs, b_obs, W_nei, b_nei, W_mode, b_mode):
    obs_p = obs.reshape(B, D_IN)
    neis_p = neis.reshape(B * N, D_IN)
    WoT = jnp.transpose(W_obs, (2, 0, 1)).reshape(D_IN, NUM_CLASS * EMBED)
    WnT = (jnp.transpose(W_nei, (2, 0, 1))
           .reshape(D_IN, (NUM_CLASS + 1) * EMBED).astype(jnp.bfloat16))
    W1 = W_mode[:, :EMBED]
    W2 = W_mode[:, EMBED:]
    modes_flat = modes.reshape(NUM_CLASS * NUM_MODES, EMBED)
    b_mode2 = b_mode.reshape(1, EMBED)
    lbl_a = self_labels.reshape(B // _BLK_A, _BLK_A, 1)
    lbl_b = nei_labels.reshape(B * N // _BLK_B, _BLK_B, 1)

    n_a = B // _BLK_A
    x_out = pl.pallas_call(
        _self_mode_body,
        grid=(n_a,),
        in_specs=[
            pl.BlockSpec((_BLK_A, D_IN), lambda i: (i, 0)),
            pl.BlockSpec((1, _BLK_A, 1), lambda i: (i, 0, 0)),
            pl.BlockSpec((D_IN, NUM_CLASS * EMBED), lambda i: (0, 0)),
            pl.BlockSpec((NUM_CLASS, EMBED), lambda i: (0, 0)),
            pl.BlockSpec((EMBED, EMBED), lambda i: (0, 0)),
            pl.BlockSpec((EMBED, EMBED), lambda i: (0, 0)),
            pl.BlockSpec((NUM_CLASS * NUM_MODES, EMBED), lambda i: (0, 0)),
            pl.BlockSpec((1, EMBED), lambda i: (0, 0)),
        ],
        out_specs=pl.BlockSpec((_BLK_A, NUM_MODES, EMBED), lambda i: (i, 0, 0)),
        out_shape=jax.ShapeDtypeStruct((B, NUM_MODES, EMBED), jnp.float32),
    )(obs_p, lbl_a, WoT, b_obs, W1, W2, modes_flat, b_mode2)

    n_b = B * N // _BLK_B
    nei_out = pl.pallas_call(
        _nei_body,
        grid=(n_b,),
        in_specs=[
            pl.BlockSpec((_BLK_B, D_IN), lambda i: (i, 0)),
            pl.BlockSpec((1, _BLK_B, 1), lambda i: (i, 0, 0)),
            pl.BlockSpec((D_IN, (NUM_CLASS + 1) * EMBED), lambda i: (0, 0)),
            pl.BlockSpec((NUM_CLASS + 1, EMBED), lambda i: (0, 0)),
        ],
        out_specs=pl.BlockSpec((_BLK_B, EMBED), lambda i: (i, 0)),
        out_shape=jax.ShapeDtypeStruct((B * N, EMBED), jnp.float32),
    )(neis_p, lbl_b, WnT, b_nei)

    return (x_out, nei_out.reshape(B, N, EMBED))


# speedup vs baseline: 1.9211x; 1.0751x over previous
import jax
import jax.numpy as jnp
from jax import lax
from jax.experimental import pallas as pl

NUM_CLASS = 8
IN_SIZE = 2
OBS_LEN = 50
EMBED = 256
B = 1024
N = 32
NUM_MODES = 20
D_IN = IN_SIZE * OBS_LEN
D_PAD = 128

_BLK_A = 256
_BLK_B = 1024


def _self_mode_body(obs_ref, lbl_ref, Wo_ref, b_obs_ref, W1_ref, W2_ref,
                    modes_ref, b_mode_ref, out_ref):
    lbl = lbl_ref[0, :, :]
    y_all = lax.dot_general(obs_ref[...], Wo_ref[...], (((1,), (1,)), ((), ())),
                            preferred_element_type=jnp.float32)
    x = jnp.zeros((_BLK_A, EMBED), jnp.float32)
    for c in range(NUM_CLASS):
        feat = y_all[:, c * EMBED:(c + 1) * EMBED] + b_obs_ref[c, :][None, :]
        x = jnp.where(lbl == c, feat, x)
    xw1 = lax.dot_general(x, W1_ref[...], (((1,), (1,)), ((), ())),
                          preferred_element_type=jnp.float32)
    m2 = lax.dot_general(modes_ref[...], W2_ref[...], (((1,), (1,)), ((), ())),
                         preferred_element_type=jnp.float32)
    m2r = m2.reshape(NUM_CLASS, NUM_MODES * EMBED)
    onehot = (lbl ==
              lax.broadcasted_iota(jnp.int32, (_BLK_A, NUM_CLASS), 1)
              ).astype(jnp.float32)
    mode_part = jnp.dot(onehot, m2r,
                        preferred_element_type=jnp.float32)
    out = (mode_part.reshape(_BLK_A, NUM_MODES, EMBED)
           + xw1[:, None, :] + b_mode_ref[0, :][None, None, :])
    out_ref[...] = out


def _nei_body(neis_ref, lbl_ref, Wn_ref, b_nei_ref, out_ref):
    v = neis_ref[...]
    t = jnp.where(v >= 0, 1.0 / (v + 0.0001), 1.0 / (v - 0.0001))
    p = lax.dot_general(t.astype(jnp.bfloat16), Wn_ref[...].astype(jnp.bfloat16),
                        (((1,), (1,)), ((), ())),
                        preferred_element_type=jnp.float32)
    lbl = lbl_ref[0, :, :]
    out = jnp.zeros((_BLK_B, EMBED), jnp.float32)
    for c in range(NUM_CLASS + 1):
        feat = p[:, c * EMBED:(c + 1) * EMBED] + b_nei_ref[c, :][None, :]
        out = jnp.where(lbl == c, feat, out)
    out_ref[...] = out


def kernel(obs, neis, self_labels, nei_labels, modes,
           W_obs, b_obs, W_nei, b_nei, W_mode, b_mode):
    obs_p = obs.reshape(B, D_IN)
    neis_p = neis.reshape(B * N, D_IN)
    Wo = W_obs.reshape(NUM_CLASS * EMBED, D_IN)
    Wn = W_nei.reshape((NUM_CLASS + 1) * EMBED, D_IN)
    W1 = W_mode[:, :EMBED]
    W2 = W_mode[:, EMBED:]
    modes_flat = modes.reshape(NUM_CLASS * NUM_MODES, EMBED)
    b_mode2 = b_mode.reshape(1, EMBED)
    lbl_a = self_labels.reshape(B // _BLK_A, _BLK_A, 1)
    lbl_b = nei_labels.reshape(B * N // _BLK_B, _BLK_B, 1)

    n_a = B // _BLK_A
    x_out = pl.pallas_call(
        _self_mode_body,
        grid=(n_a,),
        in_specs=[
            pl.BlockSpec((_BLK_A, D_IN), lambda i: (i, 0)),
            pl.BlockSpec((1, _BLK_A, 1), lambda i: (i, 0, 0)),
            pl.BlockSpec((NUM_CLASS * EMBED, D_IN), lambda i: (0, 0)),
            pl.BlockSpec((NUM_CLASS, EMBED), lambda i: (0, 0)),
            pl.BlockSpec((EMBED, EMBED), lambda i: (0, 0)),
            pl.BlockSpec((EMBED, EMBED), lambda i: (0, 0)),
            pl.BlockSpec((NUM_CLASS * NUM_MODES, EMBED), lambda i: (0, 0)),
            pl.BlockSpec((1, EMBED), lambda i: (0, 0)),
        ],
        out_specs=pl.BlockSpec((_BLK_A, NUM_MODES, EMBED), lambda i: (i, 0, 0)),
        out_shape=jax.ShapeDtypeStruct((B, NUM_MODES, EMBED), jnp.float32),
    )(obs_p, lbl_a, Wo, b_obs, W1, W2, modes_flat, b_mode2)

    n_b = B * N // _BLK_B
    nei_out = pl.pallas_call(
        _nei_body,
        grid=(n_b,),
        in_specs=[
            pl.BlockSpec((_BLK_B, D_IN), lambda i: (i, 0)),
            pl.BlockSpec((1, _BLK_B, 1), lambda i: (i, 0, 0)),
            pl.BlockSpec(((NUM_CLASS + 1) * EMBED, D_IN), lambda i: (0, 0)),
            pl.BlockSpec((NUM_CLASS + 1, EMBED), lambda i: (0, 0)),
        ],
        out_specs=pl.BlockSpec((_BLK_B, EMBED), lambda i: (i, 0)),
        out_shape=jax.ShapeDtypeStruct((B * N, EMBED), jnp.float32),
    )(neis_p, lbl_b, Wn, b_nei)

    return (x_out, nei_out.reshape(B, N, EMBED))
